# Initial kernel scaffold; baseline (speedup 1.0000x reference)
import jax
import jax.numpy as jnp
from jax.experimental import pallas as pl


def kernel(Q, AT, b, c, x, y, Iy, il, iu, l, u):
    raise NotImplementedError("write your pallas kernel here")


# speedup vs baseline: 1.0958x; 1.0958x over previous
import jax
import jax.numpy as jnp
from jax.experimental import pallas as pl
from jax.experimental.pallas import tpu as pltpu

N = 4096
BM = 256


def _body(q_ref, at_ref, xt_ref, yt_ref, c_ref, out_ref, gmax_ref, cmax_ref):
    i = pl.program_id(0)
    qx = jnp.sum(q_ref[...] * xt_ref[...], axis=1, keepdims=True)
    aty = jnp.sum(at_ref[...] * yt_ref[...], axis=1, keepdims=True)
    pg = qx + aty + c_ref[...]
    m = jnp.max(jnp.abs(pg))
    mc = jnp.max(jnp.abs(c_ref[...]))

    @pl.when(i == 0)
    def _init():
        gmax_ref[0, 0] = m
        cmax_ref[0, 0] = mc

    @pl.when(i > 0)
    def _acc():
        gmax_ref[0, 0] = jnp.maximum(gmax_ref[0, 0], m)
        cmax_ref[0, 0] = jnp.maximum(cmax_ref[0, 0], mc)

    @pl.when(i == pl.num_programs(0) - 1)
    def _fin():
        out_ref[0, 0] = gmax_ref[0, 0] / (1.0 + cmax_ref[0, 0])


def kernel(Q, AT, b, c, x, y, Iy, il, iu, l, u):
    xt = x.reshape(1, N)
    yt = y.reshape(1, N)
    c2 = c.reshape(N, 1)
    grid = N // BM
    out = pl.pallas_call(
        _body,
        grid=(grid,),
        in_specs=[
            pl.BlockSpec((BM, N), lambda i: (i, 0)),
            pl.BlockSpec((BM, N), lambda i: (i, 0)),
            pl.BlockSpec((1, N), lambda i: (0, 0)),
            pl.BlockSpec((1, N), lambda i: (0, 0)),
            pl.BlockSpec((BM, 1), lambda i: (i, 0)),
        ],
        out_specs=pl.BlockSpec(memory_space=pltpu.SMEM),
        out_shape=jax.ShapeDtypeStruct((1, 1), jnp.float32),
        scratch_shapes=[
            pltpu.SMEM((1, 1), jnp.float32),
            pltpu.SMEM((1, 1), jnp.float32),
        ],
    )(Q, AT, xt, yt, c2)
    return out[0, 0]
